# Initial kernel scaffold; baseline (speedup 1.0000x reference)
import jax
import jax.numpy as jnp
from jax.experimental import pallas as pl


def kernel(inputs, rel_embeddings):
    raise NotImplementedError("write your pallas kernel here")


# speedup vs baseline: 6.2128x; 6.2128x over previous
import functools

import jax
import jax.numpy as jnp
from jax import lax
from jax.experimental import pallas as pl
from jax.experimental.pallas import tpu as pltpu
from jax.experimental.pallas import tpu_sc as plsc

_MAX_POSITION = 2048
_DEPTH = 64
_LENGTH = 1024

_NW = 32
_ROWS_PER_W = _LENGTH // _NW
_CHUNK = 128
_NCHUNK = 9
_STAGE = _CHUNK * _NCHUNK


def _sc_body(rel_hbm, out_hbm, stage_v, idx_v, sem):
    nc = 2
    wid = lax.axis_index("s") * nc + lax.axis_index("c")
    base = wid * _ROWS_PER_W

    lanes = lax.iota(jnp.int32, 16)
    top = 2078 + base
    for j in range(_NCHUNK):
        for c in range(_CHUNK // 16):
            start = top - j * _CHUNK - c * 16
            idx_v[j, pl.ds(c * 16, 16)] = start - lanes

    copies = [
        pltpu.async_copy(
            rel_hbm.at[idx_v.at[j]],
            stage_v.at[pl.ds(j * _CHUNK, _CHUNK)],
            sem,
        )
        for j in range(_NCHUNK)
    ]
    for cp in copies:
        cp.wait()

    for r in range(_ROWS_PER_W):
        pltpu.sync_copy(
            stage_v.at[pl.ds(_ROWS_PER_W - 1 - r, _LENGTH)],
            out_hbm.at[base + r],
        )


@jax.jit
def _rel_pos_sc(rel_embeddings):
    mesh = plsc.VectorSubcoreMesh(core_axis_name="c", subcore_axis_name="s")
    return pl.kernel(
        _sc_body,
        out_type=jax.ShapeDtypeStruct((_LENGTH, _LENGTH, _DEPTH), jnp.float32),
        mesh=mesh,
        scratch_types=[
            pltpu.VMEM((_STAGE, _DEPTH), jnp.float32),
            pltpu.VMEM((_NCHUNK, _CHUNK), jnp.int32),
            pltpu.SemaphoreType.DMA,
        ],
        compiler_params=pltpu.CompilerParams(use_tc_tiling_on_sc=False),
    )(rel_embeddings)


def kernel(inputs, rel_embeddings):
    del inputs
    return _rel_pos_sc(rel_embeddings)
